# Initial kernel scaffold; baseline (speedup 1.0000x reference)
import functools

import jax
import jax.numpy as jnp
from jax import lax
from jax.experimental import pallas as pl
from jax.experimental.pallas import tpu as pltpu
from jax.experimental.pallas import tpu_sc as plsc

_NT = 32
_NS = 16
_B = 128
_K = 4


def _make_sc_agg(n_nodes, n_acc, ep, d, with_deg):
    per_tile = ep // _NT
    kc = per_tile // (_K * _B)
    zrows = n_acc // _NS
    orows = n_nodes // _NS
    mesh = plsc.VectorSubcoreMesh(core_axis_name="c", subcore_axis_name="s")

    out_type = [jax.ShapeDtypeStruct((2, n_nodes, d), jnp.float32)]
    scratch = [
        pltpu.VMEM((_K, _B), jnp.int32),
        pltpu.VMEM((_K, _B), jnp.int32),
        pltpu.VMEM((_K * _B, d), jnp.float32),
        pltpu.VMEM((128, d), jnp.float32),
        pltpu.VMEM_SHARED((n_acc, d), jnp.float32),
        pltpu.SemaphoreType.DMA,
    ]
    if with_deg:
        out_type.append(jax.ShapeDtypeStruct((2, n_nodes, 16), jnp.float32))
        scratch += [
            pltpu.VMEM((_B, 16), jnp.float32),
            pltpu.VMEM((128, 16), jnp.float32),
            pltpu.VMEM_SHARED((n_acc, 16), jnp.float32),
        ]

    def body(src_hbm, dst_hbm, feat_hbm, *refs):
        if with_deg:
            (out_acc, out_deg, srcb, dstb, rows, zb, acc, sem,
             tpl, zd, dacc) = refs
        else:
            out_acc, srcb, dstb, rows, zb, acc, sem = refs
        c = lax.axis_index("c")
        s = lax.axis_index("s")
        wid = s * 2 + c

        zvec = jnp.zeros((16,), jnp.float32)

        def zrow(r, _):
            def zcol(cc, _):
                zb[r, pl.ds(cc * 16, 16)] = zvec
                return 0
            return lax.fori_loop(0, d // 16, zcol, 0)

        lax.fori_loop(0, 128, zrow, 0)
        zbase = s * zrows
        for i in range(zrows // 128):
            pltpu.sync_copy(zb, acc.at[pl.ds(zbase + i * 128, 128)])

        if with_deg:
            idx16 = lax.iota(jnp.int32, 16)
            onehot = jnp.where(idx16 == 0, 1.0, 0.0).astype(jnp.float32)

            def trow(r, _):
                tpl[r, pl.ds(0, 16)] = onehot
                zd[r, pl.ds(0, 16)] = zvec
                return 0

            lax.fori_loop(0, 128, trow, 0)
            for i in range(zrows // 128):
                pltpu.sync_copy(zd, dacc.at[pl.ds(zbase + i * 128, 128)])

        plsc.subcore_barrier()

        def chunk(i, _):
            pltpu.sync_copy(src_hbm.at[wid, pl.ds(i * _K, _K)], srcb)
            pltpu.sync_copy(dst_hbm.at[wid, pl.ds(i * _K, _K)], dstb)
            cps = [
                pltpu.async_copy(
                    feat_hbm.at[srcb.at[j]], rows.at[pl.ds(j * _B, _B)], sem
                )
                for j in range(_K)
            ]
            for cp in cps:
                cp.wait()
            for j in range(_K):
                pltpu.sync_copy(
                    rows.at[pl.ds(j * _B, _B)], acc.at[dstb.at[j]], add=True
                )
            if with_deg:
                for j in range(_K):
                    pltpu.sync_copy(tpl, dacc.at[dstb.at[j]], add=True)
            return 0

        lax.fori_loop(0, kc, chunk, 0)

        plsc.subcore_barrier()

        ob = s * orows
        pltpu.sync_copy(acc.at[pl.ds(ob, orows)], out_acc.at[c, pl.ds(ob, orows)])
        if with_deg:
            pltpu.sync_copy(dacc.at[pl.ds(ob, orows)], out_deg.at[c, pl.ds(ob, orows)])

    return pl.kernel(body, out_type=out_type, mesh=mesh, scratch_types=scratch)


def _tc_layer(acc, deg, x, wl_t, bl, wr_t, relu):
    n, d = x.shape
    rb = 2000
    assert n % rb == 0

    def body(acc_ref, deg_ref, x_ref, wl_ref, b_ref, wr_ref, o_ref):
        a = acc_ref[0] + acc_ref[1]
        dg = deg_ref[0] + deg_ref[1]
        mean = a / jnp.maximum(dg[:, 0:1], 1.0)
        out = jnp.dot(mean, wl_ref[...], preferred_element_type=jnp.float32)
        out = out + b_ref[...]
        out = out + jnp.dot(x_ref[...], wr_ref[...], preferred_element_type=jnp.float32)
        if relu:
            out = jnp.maximum(out, 0.0)
        o_ref[...] = out

    return pl.pallas_call(
        body,
        grid=(n // rb,),
        in_specs=[
            pl.BlockSpec((2, rb, d), lambda i: (0, i, 0)),
            pl.BlockSpec((2, rb, 16), lambda i: (0, i, 0)),
            pl.BlockSpec((rb, d), lambda i: (i, 0)),
            pl.BlockSpec((d, d), lambda i: (0, 0)),
            pl.BlockSpec((1, d), lambda i: (0, 0)),
            pl.BlockSpec((d, d), lambda i: (0, 0)),
        ],
        out_specs=pl.BlockSpec((rb, d), lambda i: (i, 0)),
        out_shape=jax.ShapeDtypeStruct((n, d), jnp.float32),
    )(acc, deg, x, wl_t, bl, wr_t)


def kernel(x, edge_index, W1_l, b1_l, W1_r, W2_l, b2_l, W2_r):
    n, d = x.shape
    e = edge_index.shape[1]
    chunk_e = _NT * _K * _B
    ep = chunk_e * (-(-e // chunk_e))
    n_acc = (_NS * 128) * (-(-(n + 1) // (_NS * 128)))

    src = edge_index[0]
    dst = edge_index[1]
    pad = ep - e
    if pad:
        src = jnp.concatenate([src, jnp.zeros((pad,), src.dtype)])
        dst = jnp.concatenate([dst, jnp.full((pad,), n, dst.dtype)])
    srcr = src.reshape(_NT, -1, _B)
    dstr = dst.reshape(_NT, -1, _B)

    acc1, deg = _make_sc_agg(n, n_acc, ep, d, True)(srcr, dstr, x)
    h = _tc_layer(acc1, deg, x, W1_l.T, b1_l.reshape(1, -1), W1_r.T, True)
    acc2 = _make_sc_agg(n, n_acc, ep, d, False)(srcr, dstr, h)
    if isinstance(acc2, (tuple, list)):
        acc2 = acc2[0]
    return _tc_layer(acc2, deg, h, W2_l.T, b2_l.reshape(1, -1), W2_r.T, False)


# speedup vs baseline: 3.8565x; 3.8565x over previous
import jax
import jax.numpy as jnp
from jax import lax
from jax.experimental import pallas as pl
from jax.experimental.pallas import tpu as pltpu
from jax.experimental.pallas import tpu_sc as plsc

_NS = 16
_B = 128
_K = 4
_DH = 64


def _make_sc_agg(n_nodes, n_acc, ep, with_deg):
    per_tile = ep // _NS
    kc = per_tile // (_K * _B)
    zrows = n_acc // _NS
    mesh = plsc.VectorSubcoreMesh(core_axis_name="c", subcore_axis_name="s")

    out_type = [jax.ShapeDtypeStruct((2, n_acc, _DH), jnp.float32)]
    scratch = [
        pltpu.VMEM((_K, _B), jnp.int32),
        pltpu.VMEM((_K, _B), jnp.int32),
        pltpu.VMEM((_K * _B, _DH), jnp.float32),
        pltpu.VMEM((128, _DH), jnp.float32),
        pltpu.VMEM_SHARED((n_acc, _DH), jnp.float32),
        pltpu.SemaphoreType.DMA,
    ]
    if with_deg:
        out_type.append(jax.ShapeDtypeStruct((2, n_acc, 16), jnp.float32))
        scratch += [
            pltpu.VMEM((_B, 16), jnp.float32),
            pltpu.VMEM((128, 16), jnp.float32),
            pltpu.VMEM_SHARED((n_acc, 16), jnp.float32),
        ]

    def body(src_hbm, dst_hbm, feat_hbm, *refs):
        if with_deg:
            (out_acc, out_deg, srcb, dstb, rows, zb, acc, sem,
             tpl, zd, dacc) = refs
        else:
            out_acc, srcb, dstb, rows, zb, acc, sem = refs
        c = lax.axis_index("c")
        s = lax.axis_index("s")

        zvec = jnp.zeros((16,), jnp.float32)

        def zrow(r, _):
            def zcol(cc, _):
                zb[r, pl.ds(cc * 16, 16)] = zvec
                return 0
            return lax.fori_loop(0, _DH // 16, zcol, 0)

        lax.fori_loop(0, 128, zrow, 0)
        zbase = s * zrows
        for i in range(zrows // 128):
            pltpu.sync_copy(zb, acc.at[pl.ds(zbase + i * 128, 128)])

        if with_deg:
            idx16 = lax.iota(jnp.int32, 16)
            half1 = jnp.where(idx16 == 0, 0.5, 0.0).astype(jnp.float32)

            def trow(r, _):
                tpl[r, pl.ds(0, 16)] = half1
                zd[r, pl.ds(0, 16)] = zvec
                return 0

            lax.fori_loop(0, 128, trow, 0)
            for i in range(zrows // 128):
                pltpu.sync_copy(zd, dacc.at[pl.ds(zbase + i * 128, 128)])

        plsc.subcore_barrier()

        def chunk(i, _):
            pltpu.sync_copy(src_hbm.at[c, s, pl.ds(i * _K, _K)], srcb)
            pltpu.sync_copy(dst_hbm.at[s, pl.ds(i * _K, _K)], dstb)
            cps = [
                pltpu.async_copy(
                    feat_hbm.at[srcb.at[j]], rows.at[pl.ds(j * _B, _B)], sem
                )
                for j in range(_K)
            ]
            for cp in cps:
                cp.wait()
            for j in range(_K):
                pltpu.sync_copy(
                    rows.at[pl.ds(j * _B, _B)], acc.at[dstb.at[j]], add=True
                )
            if with_deg:
                for j in range(_K):
                    pltpu.sync_copy(tpl, dacc.at[dstb.at[j]], add=True)
            return 0

        lax.fori_loop(0, kc, chunk, 0)

        plsc.subcore_barrier()

        ob = s * zrows
        pltpu.sync_copy(acc.at[pl.ds(ob, zrows)], out_acc.at[c, pl.ds(ob, zrows)])
        if with_deg:
            pltpu.sync_copy(dacc.at[pl.ds(ob, zrows)], out_deg.at[c, pl.ds(ob, zrows)])

    return pl.kernel(
        body, out_type=out_type, mesh=mesh, scratch_types=scratch,
        compiler_params=pltpu.CompilerParams(use_tc_tiling_on_sc=False),
    )


def _tc_layer(acc, deg, x, wl_t, bl, wr_t, relu):
    n, d = x.shape
    rb = 2000
    assert n % rb == 0

    def body(acc_ref, deg_ref, x_ref, wl_ref, b_ref, wr_ref, o_ref):
        dg = jnp.maximum(deg_ref[0][:, 0:1] + deg_ref[1][:, 0:1], 1.0)
        out = jnp.dot(acc_ref[0] / dg, wl_ref[0:_DH, :],
                      preferred_element_type=jnp.float32)
        out += jnp.dot(acc_ref[1] / dg, wl_ref[_DH:, :],
                       preferred_element_type=jnp.float32)
        out += b_ref[...]
        out += jnp.dot(x_ref[...], wr_ref[...], preferred_element_type=jnp.float32)
        if relu:
            out = jnp.maximum(out, 0.0)
        o_ref[...] = out

    return pl.pallas_call(
        body,
        grid=(n // rb,),
        in_specs=[
            pl.BlockSpec((2, rb, _DH), lambda i: (0, i, 0)),
            pl.BlockSpec((2, rb, 16), lambda i: (0, i, 0)),
            pl.BlockSpec((rb, d), lambda i: (i, 0)),
            pl.BlockSpec((d, d), lambda i: (0, 0)),
            pl.BlockSpec((1, d), lambda i: (0, 0)),
            pl.BlockSpec((d, d), lambda i: (0, 0)),
        ],
        out_specs=pl.BlockSpec((rb, d), lambda i: (i, 0)),
        out_shape=jax.ShapeDtypeStruct((n, d), jnp.float32),
    )(acc, deg, x, wl_t, bl, wr_t)


def _split_cols(f):
    return jnp.concatenate([f[:, :_DH], f[:, _DH:]], axis=0)


def kernel(x, edge_index, W1_l, b1_l, W1_r, W2_l, b2_l, W2_r):
    n, d = x.shape
    e = edge_index.shape[1]
    chunk_e = _NS * _K * _B
    ep = chunk_e * (-(-e // chunk_e))
    n_acc = (_NS * 128) * (-(-(n + 1) // (_NS * 128)))

    src = edge_index[0]
    dst = edge_index[1]
    pad = ep - e
    if pad:
        src = jnp.concatenate([src, jnp.zeros((pad,), src.dtype)])
        dst = jnp.concatenate([dst, jnp.full((pad,), n, dst.dtype)])
    srcr = src.reshape(_NS, -1, _B)
    srcr2 = jnp.stack([srcr, srcr + n])
    dstr = dst.reshape(_NS, -1, _B)

    acc1, deg = _make_sc_agg(n, n_acc, ep, True)(srcr2, dstr, _split_cols(x))
    h = _tc_layer(acc1, deg, x, W1_l.T, b1_l.reshape(1, -1), W1_r.T, True)
    acc2 = _make_sc_agg(n, n_acc, ep, False)(srcr2, dstr, _split_cols(h))
    if isinstance(acc2, (tuple, list)):
        acc2 = acc2[0]
    return _tc_layer(acc2, deg, h, W2_l.T, b2_l.reshape(1, -1), W2_r.T, False)


# speedup vs baseline: 4.1625x; 1.0793x over previous
import jax
import jax.numpy as jnp
from jax import lax
from jax.experimental import pallas as pl
from jax.experimental.pallas import tpu as pltpu
from jax.experimental.pallas import tpu_sc as plsc

_NS = 16
_B = 128
_K = 4
_DH = 64


import functools


@functools.lru_cache(maxsize=None)
def _make_sc_agg(n_nodes, n_acc, ep, with_deg):
    per_tile = ep // _NS
    kc = per_tile // (_K * _B)
    zrows = n_acc // _NS
    mesh = plsc.VectorSubcoreMesh(core_axis_name="c", subcore_axis_name="s")

    rows_pt = per_tile // _B
    ih = rows_pt // 2
    out_type = [jax.ShapeDtypeStruct((2, n_acc, _DH), jnp.float32)]
    scratch = [
        pltpu.VMEM((ih, _B), jnp.int32),
        pltpu.VMEM((ih, _B), jnp.int32),
        pltpu.VMEM((_K * _B, _DH), jnp.float32),
        pltpu.VMEM((128, _DH), jnp.float32),
        pltpu.VMEM_SHARED((n_acc, _DH), jnp.float32),
        pltpu.SemaphoreType.DMA,
    ]
    if with_deg:
        out_type.append(jax.ShapeDtypeStruct((2, n_acc, 16), jnp.float32))
        scratch += [
            pltpu.VMEM((_B, 16), jnp.float32),
            pltpu.VMEM((128, 16), jnp.float32),
            pltpu.VMEM_SHARED((n_acc, 16), jnp.float32),
        ]

    def body(src_hbm, dst_hbm, feat_hbm, *refs):
        if with_deg:
            (out_acc, out_deg, srca, dsta, rows0, zb, acc, sem_g,
             tpl, zd, dacc) = refs
        else:
            out_acc, srca, dsta, rows0, zb, acc, sem_g = refs
        c = lax.axis_index("c")
        s = lax.axis_index("s")

        zvec = jnp.zeros((16,), jnp.float32)

        def zrow(r, _):
            def zcol(cc, _):
                zb[r, pl.ds(cc * 16, 16)] = zvec
                return 0
            return lax.fori_loop(0, _DH // 16, zcol, 0)

        lax.fori_loop(0, 128, zrow, 0)
        zbase = s * zrows
        for i in range(zrows // 128):
            pltpu.sync_copy(zb, acc.at[pl.ds(zbase + i * 128, 128)])

        if with_deg:
            idx16 = lax.iota(jnp.int32, 16)
            half1 = jnp.where(idx16 == 0, 0.5, 0.0).astype(jnp.float32)

            def trow(r, _):
                tpl[r, pl.ds(0, 16)] = half1
                zd[r, pl.ds(0, 16)] = zvec
                return 0

            lax.fori_loop(0, 128, trow, 0)
            for i in range(zrows // 128):
                pltpu.sync_copy(zd, dacc.at[pl.ds(zbase + i * 128, 128)])

        plsc.subcore_barrier()

        def chunk(i, _):
            for cp in [
                pltpu.async_copy(
                    feat_hbm.at[srca.at[i * _K + j]],
                    rows0.at[pl.ds(j * _B, _B)], sem_g,
                )
                for j in range(_K)
            ]:
                cp.wait()
            for j in range(_K):
                pltpu.sync_copy(
                    rows0.at[pl.ds(j * _B, _B)], acc.at[dsta.at[i * _K + j]],
                    add=True,
                )
            if with_deg:
                for j in range(_K):
                    pltpu.sync_copy(tpl, dacc.at[dsta.at[i * _K + j]], add=True)
            return 0

        for h in range(2):
            pltpu.sync_copy(src_hbm.at[c, s, pl.ds(h * ih, ih)], srca)
            pltpu.sync_copy(dst_hbm.at[s, pl.ds(h * ih, ih)], dsta)
            lax.fori_loop(0, kc // 2, chunk, 0)

        plsc.subcore_barrier()

        ob = s * zrows
        pltpu.sync_copy(acc.at[pl.ds(ob, zrows)], out_acc.at[c, pl.ds(ob, zrows)])
        if with_deg:
            pltpu.sync_copy(dacc.at[pl.ds(ob, zrows)], out_deg.at[c, pl.ds(ob, zrows)])

    return pl.kernel(
        body, out_type=out_type, mesh=mesh, scratch_types=scratch,
        compiler_params=pltpu.CompilerParams(use_tc_tiling_on_sc=False),
    )


def _tc_layer(acc, deg, x, wl_t, bl, wr_t, relu):
    n, d = x.shape
    rb = 2000
    assert n % rb == 0

    def body(acc_ref, deg_ref, x_ref, wl_ref, b_ref, wr_ref, o_ref):
        dg = jnp.maximum(deg_ref[0][:, 0:1] + deg_ref[1][:, 0:1], 1.0)
        out = jnp.dot(acc_ref[0] / dg, wl_ref[0:_DH, :],
                      preferred_element_type=jnp.float32)
        out += jnp.dot(acc_ref[1] / dg, wl_ref[_DH:, :],
                       preferred_element_type=jnp.float32)
        out += b_ref[...]
        out += jnp.dot(x_ref[...], wr_ref[...], preferred_element_type=jnp.float32)
        if relu:
            out = jnp.maximum(out, 0.0)
        o_ref[...] = out

    return pl.pallas_call(
        body,
        grid=(n // rb,),
        in_specs=[
            pl.BlockSpec((2, rb, _DH), lambda i: (0, i, 0)),
            pl.BlockSpec((2, rb, 16), lambda i: (0, i, 0)),
            pl.BlockSpec((rb, d), lambda i: (i, 0)),
            pl.BlockSpec((d, d), lambda i: (0, 0)),
            pl.BlockSpec((1, d), lambda i: (0, 0)),
            pl.BlockSpec((d, d), lambda i: (0, 0)),
        ],
        out_specs=pl.BlockSpec((rb, d), lambda i: (i, 0)),
        out_shape=jax.ShapeDtypeStruct((n, d), jnp.float32),
    )(acc, deg, x, wl_t, bl, wr_t)


def _split_cols(f):
    return jnp.concatenate([f[:, :_DH], f[:, _DH:]], axis=0)


def kernel(x, edge_index, W1_l, b1_l, W1_r, W2_l, b2_l, W2_r):
    n, d = x.shape
    e = edge_index.shape[1]
    chunk_e = _NS * _K * _B * 2
    ep = chunk_e * (-(-e // chunk_e))
    n_acc = (_NS * 128) * (-(-(n + 1) // (_NS * 128)))

    src = edge_index[0]
    dst = edge_index[1]
    pad = ep - e
    if pad:
        src = jnp.concatenate([src, jnp.zeros((pad,), src.dtype)])
        dst = jnp.concatenate([dst, jnp.full((pad,), n, dst.dtype)])
    srcr = src.reshape(_NS, -1, _B)
    srcr2 = jnp.stack([srcr, srcr + n])
    dstr = dst.reshape(_NS, -1, _B)

    acc1, deg = _make_sc_agg(n, n_acc, ep, True)(srcr2, dstr, _split_cols(x))
    h = _tc_layer(acc1, deg, x, W1_l.T, b1_l.reshape(1, -1), W1_r.T, True)
    acc2 = _make_sc_agg(n, n_acc, ep, False)(srcr2, dstr, _split_cols(h))
    if isinstance(acc2, (tuple, list)):
        acc2 = acc2[0]
    return _tc_layer(acc2, deg, h, W2_l.T, b2_l.reshape(1, -1), W2_r.T, False)
